# baseline (speedup 1.0000x reference)
import functools

import jax
import jax.numpy as jnp
from jax.experimental import pallas as pl
from jax.experimental.pallas import tpu as pltpu

NBINS = 16
B = 16384
F = 100
OUT_D = F * NBINS


def _quantile_rows():
    rows = []
    for t in range(NBINS):
        p = t * (B - 1) / (NBINS - 1)
        lo = int(p)
        frac = p - lo
        hi = min(lo + 1, B - 1)
        rows.append((lo, hi, frac))
    return rows


C = 2048
NCHUNK = B // C


def _stage_tables():
    ks, ds = [], []
    k = 2
    while k <= B:
        d = k // 2
        while d >= 1:
            ks.append(k)
            ds.append(d)
            d //= 2
        k *= 2
    return ks, ds


_KS, _DS = _stage_tables()
NSTAGES = len(_KS)


def _sort_bins_kernel(ks_ref, ds_ref, x_ref, bins_ref, xs_ref):
    xs_ref[...] = x_ref[...]

    def stage(s, carry):
        k = ks_ref[s]
        d = ds_ref[s]

        def in_chunk():
            def chunk_body(c, _):
                base = c * C
                xc = xs_ref[pl.ds(base, C), :]
                gid = jax.lax.broadcasted_iota(jnp.int32, (C, F), 0) + base
                up = (gid & d) == 0
                asc = (gid & k) == 0
                take_min = up == asc
                partner = jnp.where(up, pltpu.roll(xc, C - d, 0),
                                    pltpu.roll(xc, d, 0))
                xs_ref[pl.ds(base, C), :] = jnp.where(
                    take_min, jnp.minimum(xc, partner),
                    jnp.maximum(xc, partner))
                return 0

            jax.lax.fori_loop(0, NCHUNK, chunk_body, 0)

        def cross():
            dc = d // C

            def pair_body(p, _):
                c = (p // dc) * 2 * dc + p % dc
                cp = c + dc
                a = xs_ref[pl.ds(c * C, C), :]
                b = xs_ref[pl.ds(cp * C, C), :]
                mn = jnp.minimum(a, b)
                mx = jnp.maximum(a, b)
                asc = ((c * C) & k) == 0
                xs_ref[pl.ds(c * C, C), :] = jnp.where(asc, mn, mx)
                xs_ref[pl.ds(cp * C, C), :] = jnp.where(asc, mx, mn)
                return 0

            jax.lax.fori_loop(0, NCHUNK // 2, pair_body, 0)

        jax.lax.cond(d >= C, cross, in_chunk)
        return 0

    jax.lax.fori_loop(0, NSTAGES, stage, 0)

    for t, (lo, hi, frac) in enumerate(_quantile_rows()):
        vlo = xs_ref[pl.ds(lo, 1), :]
        if frac == 0.0:
            bins_ref[pl.ds(t, 1), :] = vlo
        else:
            vhi = xs_ref[pl.ds(hi, 1), :]
            bins_ref[pl.ds(t, 1), :] = vlo * (1.0 - frac) + vhi * frac


def _encode_kernel(x_ref, s_ref, b0_ref, b1_ref, out_ref):
    xr = jnp.dot(x_ref[...], s_ref[...],
                 preferred_element_type=jnp.float32)
    b0 = b0_ref[...]
    b1 = b1_ref[...]
    recip = 1.0 / (b1 - b0 + 1e-9)
    t_lane = jax.lax.broadcasted_iota(jnp.int32, (1, OUT_D), 1) & (NBINS - 1)
    is_t0 = t_lane == 0
    is_last = t_lane == (NBINS - 1)
    interp = (xr - b0) * recip
    hi_val = jnp.where(is_last, 0.0, 1.0)
    v = jnp.where(xr >= b1, hi_val, interp)
    v = jnp.where(xr < b0, 0.0, v)
    out_ref[...] = jnp.where(is_t0, 0.0, v)


@jax.jit
def kernel(x):
    ks = jnp.asarray(_KS, dtype=jnp.int32)
    ds = jnp.asarray(_DS, dtype=jnp.int32)
    bins = pl.pallas_call(
        _sort_bins_kernel,
        in_specs=[
            pl.BlockSpec(memory_space=pltpu.SMEM),
            pl.BlockSpec(memory_space=pltpu.SMEM),
            pl.BlockSpec(memory_space=pltpu.VMEM),
        ],
        out_specs=pl.BlockSpec(memory_space=pltpu.VMEM),
        scratch_shapes=[pltpu.VMEM((B, F), jnp.float32)],
        out_shape=jax.ShapeDtypeStruct((NBINS, F), jnp.float32),
    )(ks, ds, x)
    bT = bins.T
    b1v = bT.reshape(1, OUT_D)
    b0v = jnp.roll(bT, 1, axis=1).reshape(1, OUT_D)
    s = (jnp.arange(OUT_D, dtype=jnp.int32)[None, :] // NBINS
         == jnp.arange(F, dtype=jnp.int32)[:, None]).astype(jnp.float32)
    bt = 512
    out = pl.pallas_call(
        _encode_kernel,
        grid=(B // bt,),
        in_specs=[
            pl.BlockSpec((bt, F), lambda i: (i, 0)),
            pl.BlockSpec((F, OUT_D), lambda i: (0, 0)),
            pl.BlockSpec((1, OUT_D), lambda i: (0, 0)),
            pl.BlockSpec((1, OUT_D), lambda i: (0, 0)),
        ],
        out_specs=pl.BlockSpec((bt, OUT_D), lambda i: (i, 0)),
        out_shape=jax.ShapeDtypeStruct((B, OUT_D), jnp.float32),
        compiler_params=pltpu.CompilerParams(
            dimension_semantics=("arbitrary",),
        ),
    )(x, s, b0v, b1v)
    return out
